# baseline (speedup 1.0000x reference)
import functools

import jax
import jax.numpy as jnp
from jax import lax
from jax.experimental import pallas as pl
from jax.experimental.pallas import tpu as pltpu
from jax.experimental.pallas import tpu_sc as plsc

N = 320000
B = 10000
D = 128
H = 128

K = 5000
NBLK = N // K
WIN = 320
BPAD = 10240
CH = 64
GPAD = 10240


def _sc_gather(table, idx2d):
    info = plsc.get_sparse_core_info()
    nw = info.num_cores * info.num_subcores
    b_per_w = GPAD // nw
    n_ch = b_per_w // CH
    mesh = plsc.VectorSubcoreMesh(core_axis_name="c", subcore_axis_name="s")

    @functools.partial(
        pl.kernel,
        mesh=mesh,
        out_type=jax.ShapeDtypeStruct((GPAD, D), jnp.float32),
        scratch_types=[
            pltpu.VMEM((b_per_w // 16, 16), jnp.int32),
            pltpu.VMEM((b_per_w, D), jnp.float32),
            pltpu.SemaphoreType.DMA,
        ],
    )
    def gather_kernel(table_hbm, idx_hbm, out_hbm, idx_v, rows_v, sem):
        wid = lax.axis_index("s") * info.num_cores + lax.axis_index("c")
        pltpu.sync_copy(idx_hbm.at[wid], idx_v)

        for c in range(b_per_w // 16):
            vec = idx_v[c]
            for k in range(16):
                pltpu.async_copy(
                    table_hbm.at[pl.ds(vec[k], 1)],
                    rows_v.at[pl.ds(c * 16 + k, 1)],
                    sem,
                )
        pltpu.make_async_copy(
            table_hbm.at[pl.ds(0, b_per_w)], rows_v, sem
        ).wait()
        pltpu.sync_copy(rows_v, out_hbm.at[pl.ds(wid * b_per_w, b_per_w)])

    return gather_kernel(table, idx2d)


def _vmatmul(g, wv):
    def body(g_ref, wv_ref, out_ref):
        out_ref[...] = jnp.dot(
            g_ref[...], wv_ref[...], preferred_element_type=jnp.float32
        )

    return pl.pallas_call(
        body,
        out_shape=jax.ShapeDtypeStruct((BPAD, H), jnp.float32),
    )(g, wv)


def _fused_body(starts_ref, feats_ref, seg_ref, segr_ref, v_ref, wu_ref,
                bu_ref, we_ref, out_ref, acc_ref):
    i = pl.program_id(0)

    @pl.when(i == 0)
    def _init():
        acc_ref[...] = jnp.zeros_like(acc_ref)

    feats = feats_ref[...]
    seg = seg_ref[...]
    seg_row = segr_ref[0]
    start = starts_ref[i]
    astart = (start // 8) * 8
    rel = seg - astart
    oh = (rel == lax.broadcasted_iota(jnp.int32, (K, WIN), 1)).astype(
        jnp.float32
    )
    oh_t = (
        seg_row - astart == lax.broadcasted_iota(jnp.int32, (WIN, K), 0)
    ).astype(jnp.float32)
    u = jnp.dot(feats, wu_ref[...], preferred_element_type=jnp.float32)
    u = u + bu_ref[...]
    vwin = v_ref[pl.ds(astart, WIN), :]
    vb = jnp.dot(oh, vwin, preferred_element_type=jnp.float32)
    sig = jax.nn.sigmoid(u + vb)
    e = jnp.dot(sig, we_ref[...], preferred_element_type=jnp.float32)
    w = jnp.exp(e)
    fwc = jnp.concatenate([feats * w, w], axis=1)
    part = jnp.dot(oh_t, fwc, preferred_element_type=jnp.float32)
    acc_ref[pl.ds(astart, WIN), :] += part

    @pl.when(i == NBLK - 1)
    def _finish():
        den = acc_ref[:B, D:]
        out_ref[...] = acc_ref[:B, :D] / jnp.where(den == 0.0, 1.0, den)


def _fused(feats, seg_col, seg_row3, starts, v, wu, bu_row, we_col):
    grid_spec = pltpu.PrefetchScalarGridSpec(
        num_scalar_prefetch=1,
        grid=(NBLK,),
        in_specs=[
            pl.BlockSpec((K, D), lambda i, s: (i, 0)),
            pl.BlockSpec((K, 1), lambda i, s: (i, 0)),
            pl.BlockSpec((1, 1, K), lambda i, s: (i, 0, 0)),
            pl.BlockSpec((BPAD, H), lambda i, s: (0, 0)),
            pl.BlockSpec((D, H), lambda i, s: (0, 0)),
            pl.BlockSpec((1, H), lambda i, s: (0, 0)),
            pl.BlockSpec((H, 1), lambda i, s: (0, 0)),
        ],
        out_specs=pl.BlockSpec((B, D), lambda i, s: (0, 0)),
        scratch_shapes=[pltpu.VMEM((BPAD, D + 1), jnp.float32)],
    )
    return pl.pallas_call(
        _fused_body,
        grid_spec=grid_spec,
        out_shape=jax.ShapeDtypeStruct((B, D), jnp.float32),
        compiler_params=pltpu.CompilerParams(
            dimension_semantics=("arbitrary",)
        ),
    )(starts, feats, seg_col, seg_row3, v, wu, bu_row, we_col)


def kernel(feats_s1, segment_ids, last_nodes, W_u, b_u, W_v, W_e):
    seg = segment_ids.astype(jnp.int32)
    idx = last_nodes.astype(jnp.int32)
    idx_pad = jnp.pad(idx, (0, GPAD - B)).reshape(32, GPAD // 32 // 16, 16)
    g = _sc_gather(feats_s1, idx_pad)
    v = _vmatmul(g[:BPAD], W_v)
    starts = seg[::K]
    seg_col = seg.reshape(N, 1)
    seg_row3 = seg.reshape(NBLK, 1, K)
    out = _fused(
        feats_s1, seg_col, seg_row3, starts, v, W_u,
        b_u.reshape(1, H), W_e,
    )
    return out[:, None, :]


# speedup vs baseline: 1.6930x; 1.6930x over previous
import functools

import jax
import jax.numpy as jnp
from jax import lax
from jax.experimental import pallas as pl
from jax.experimental.pallas import tpu as pltpu
from jax.experimental.pallas import tpu_sc as plsc

N = 320000
B = 10000
D = 128
H = 128

K = 6400
NBLK = N // K
SUB = 1280
NSUB = K // SUB
WIN = 136
BPAD = 10240
CH = 64
GPAD = 10240


def _sc_gather(table, idx2d):
    info = plsc.get_sparse_core_info()
    nw = info.num_cores * info.num_subcores
    b_per_w = GPAD // nw
    n_ch = b_per_w // CH
    mesh = plsc.VectorSubcoreMesh(core_axis_name="c", subcore_axis_name="s")

    @functools.partial(
        pl.kernel,
        mesh=mesh,
        out_type=jax.ShapeDtypeStruct((GPAD, D), jnp.float32),
        scratch_types=[
            pltpu.VMEM((b_per_w // 16, 16), jnp.int32),
            pltpu.VMEM((b_per_w, D), jnp.float32),
            pltpu.SemaphoreType.DMA,
        ],
    )
    def gather_kernel(table_hbm, idx_hbm, out_hbm, idx_v, rows_v, sem):
        wid = lax.axis_index("s") * info.num_cores + lax.axis_index("c")
        pltpu.sync_copy(idx_hbm.at[wid], idx_v)

        for c in range(b_per_w // 16):
            vec = idx_v[c]
            for k in range(16):
                pltpu.async_copy(
                    table_hbm.at[pl.ds(vec[k], 1)],
                    rows_v.at[pl.ds(c * 16 + k, 1)],
                    sem,
                )
        pltpu.make_async_copy(
            table_hbm.at[pl.ds(0, b_per_w)], rows_v, sem
        ).wait()
        pltpu.sync_copy(rows_v, out_hbm.at[pl.ds(wid * b_per_w, b_per_w)])

    return gather_kernel(table, idx2d)


def _vmatmul(g, wv):
    def body(g_ref, wv_ref, out_ref):
        out_ref[...] = jnp.dot(
            g_ref[...], wv_ref[...], preferred_element_type=jnp.float32
        )

    return pl.pallas_call(
        body,
        out_shape=jax.ShapeDtypeStruct((BPAD, H), jnp.float32),
    )(g, wv)


def _fused_body(starts_ref, feats_ref, seg_ref, segr_ref, v_ref, wu_ref,
                bu_ref, we_ref, out_ref, acc_ref):
    i = pl.program_id(0)

    @pl.when(i == 0)
    def _init():
        acc_ref[...] = jnp.zeros_like(acc_ref)

    feats = feats_ref[...]
    seg_col = seg_ref[...]
    seg_row = segr_ref[0]
    u_all = jnp.dot(feats, wu_ref[...], preferred_element_type=jnp.float32)
    u_all = u_all + bu_ref[...]

    for c in range(NSUB):
        lo = c * SUB
        f = feats[lo:lo + SUB]
        seg = seg_col[lo:lo + SUB]
        srow = seg_row[:, lo:lo + SUB]
        start = starts_ref[i * NSUB + c]
        astart = (start // 8) * 8
        rel = seg - astart
        oh = (rel == lax.broadcasted_iota(jnp.int32, (SUB, WIN), 1)).astype(
            jnp.float32
        )
        oh_t = (
            srow - astart == lax.broadcasted_iota(jnp.int32, (WIN, SUB), 0)
        ).astype(jnp.float32)
        vwin = v_ref[pl.ds(astart, WIN), :]
        vb = jnp.dot(oh, vwin, preferred_element_type=jnp.float32)
        sig = jax.nn.sigmoid(u_all[lo:lo + SUB] + vb)
        e = jnp.dot(sig, we_ref[...], preferred_element_type=jnp.float32)
        w = jnp.exp(e)
        fwc = jnp.concatenate([f * w, w], axis=1)
        part = jnp.dot(oh_t, fwc, preferred_element_type=jnp.float32)
        acc_ref[pl.ds(astart, WIN), :] += part

    @pl.when(i == NBLK - 1)
    def _finish():
        den = acc_ref[:B, D:]
        out_ref[...] = acc_ref[:B, :D] / jnp.where(den == 0.0, 1.0, den)


def _fused(feats, seg_col, seg_row3, starts, v, wu, bu_row, we_col):
    grid_spec = pltpu.PrefetchScalarGridSpec(
        num_scalar_prefetch=1,
        grid=(NBLK,),
        in_specs=[
            pl.BlockSpec((K, D), lambda i, s: (i, 0)),
            pl.BlockSpec((K, 1), lambda i, s: (i, 0)),
            pl.BlockSpec((1, 1, K), lambda i, s: (i, 0, 0)),
            pl.BlockSpec((BPAD, H), lambda i, s: (0, 0)),
            pl.BlockSpec((D, H), lambda i, s: (0, 0)),
            pl.BlockSpec((1, H), lambda i, s: (0, 0)),
            pl.BlockSpec((H, 1), lambda i, s: (0, 0)),
        ],
        out_specs=pl.BlockSpec((B, D), lambda i, s: (0, 0)),
        scratch_shapes=[pltpu.VMEM((BPAD, D + 1), jnp.float32)],
    )
    return pl.pallas_call(
        _fused_body,
        grid_spec=grid_spec,
        out_shape=jax.ShapeDtypeStruct((B, D), jnp.float32),
        compiler_params=pltpu.CompilerParams(
            dimension_semantics=("arbitrary",)
        ),
    )(starts, feats, seg_col, seg_row3, v, wu, bu_row, we_col)


def kernel(feats_s1, segment_ids, last_nodes, W_u, b_u, W_v, W_e):
    seg = segment_ids.astype(jnp.int32)
    idx = last_nodes.astype(jnp.int32)
    idx_pad = jnp.pad(idx, (0, GPAD - B)).reshape(32, GPAD // 32 // 16, 16)
    g = _sc_gather(feats_s1, idx_pad)
    v = _vmatmul(g[:BPAD], W_v)
    starts = seg[::SUB]
    seg_col = seg.reshape(N, 1)
    seg_row3 = seg.reshape(NBLK, 1, K)
    out = _fused(
        feats_s1, seg_col, seg_row3, starts, v, W_u,
        b_u.reshape(1, H), W_e,
    )
    return out[:, None, :]
